# baseline (speedup 1.0000x reference)
import jax
import jax.numpy as jnp
from jax import lax
from jax.experimental import pallas as pl
from jax.experimental.pallas import tpu as pltpu
from jax.experimental.pallas import tpu_sc as plsc

VOCAB = 100000
MAX_LEN = 200
EMB_DIM = 64
BATCH = 4096
SEQ_LEN = 200

NC, NS = 2, 16
NW = NC * NS
TOTAL_ROWS = BATCH * SEQ_LEN
ROWS_PER_W = TOTAL_ROWS // NW
BATCH_PER_W = BATCH // NW
CHUNK = SEQ_LEN
N_CHUNKS = BATCH_PER_W
SUB_SPLITS = [(0, 128), (128, 72)]


def _body(x_hbm, wt_hbm, pos_hbm, out_hbm,
          idx0, idx1, rows0, rows1, outv0, outv1, pos_v,
          isem0, isem1, gsem0, gsem1, osem0, osem1):
    idx = [idx0, idx1]
    rows = [rows0, rows1]
    outv = [outv0, outv1]
    isem = [isem0, isem1]
    gsem = [gsem0, gsem1]
    osem = [osem0, osem1]

    wid = lax.axis_index("s") * NC + lax.axis_index("c")
    w_base = wid * ROWS_PER_W
    w_batch = wid * BATCH_PER_W

    def gather_ops(b, issue):
        for off, n in SUB_SPLITS:
            cp = pltpu.make_async_copy(
                wt_hbm.at[idx[b].at[pl.ds(off, n)]],
                rows[b].at[pl.ds(off, n)],
                gsem[b],
            )
            if issue:
                cp.start()
            else:
                cp.wait()

    def wait_out(b):
        pltpu.make_async_copy(
            outv[b], out_hbm.at[0], osem[b]).wait()

    def add_pos(b):
        def add_body(p, carry):
            for j in range(EMB_DIM // 16):
                pv = pos_v[pl.ds(p * EMB_DIM + j * 16, 16)]
                outv[b][p, pl.ds(j * 16, 16)] = (
                    rows[b][p, pl.ds(j * 16, 16)] + pv)
            return carry
        lax.fori_loop(0, SEQ_LEN, add_body, 0)

    pltpu.sync_copy(pos_hbm, pos_v)
    pltpu.sync_copy(x_hbm.at[pl.ds(w_base, CHUNK)], idx[0])
    gather_ops(0, True)
    pltpu.async_copy(x_hbm.at[pl.ds(w_base + CHUNK, CHUNK)], idx[1], isem[1])

    def half(g, a):
        b = 1 - a

        @pl.when(g + 1 < N_CHUNKS)
        def _():
            pltpu.make_async_copy(
                x_hbm.at[pl.ds(0, CHUNK)], idx[b], isem[b]).wait()
            gather_ops(b, True)

        gather_ops(a, False)

        @pl.when(g + 2 < N_CHUNKS)
        def _():
            pltpu.async_copy(
                x_hbm.at[pl.ds(w_base + (g + 2) * CHUNK, CHUNK)],
                idx[a], isem[a])

        @pl.when(g > 1)
        def _():
            wait_out(a)

        add_pos(a)
        pltpu.async_copy(outv[a], out_hbm.at[w_batch + g], osem[a])

    def pair_body(t, carry):
        half(2 * t, 0)
        half(2 * t + 1, 1)
        return carry

    lax.fori_loop(0, N_CHUNKS // 2, pair_body, 0)
    wait_out(0)
    wait_out(1)


@jax.jit
def kernel(x, word_table, pos_table):
    x_flat = x.reshape(TOTAL_ROWS)
    pos_flat = pos_table.reshape(MAX_LEN * EMB_DIM)
    wt_wide = jnp.concatenate([word_table, word_table], axis=1)
    mesh = plsc.VectorSubcoreMesh(core_axis_name="c", subcore_axis_name="s")
    out = pl.kernel(
        _body,
        out_type=jax.ShapeDtypeStruct((BATCH, SEQ_LEN, EMB_DIM), jnp.float32),
        mesh=mesh,
        scratch_types=[
            pltpu.VMEM((CHUNK,), jnp.int32),
            pltpu.VMEM((CHUNK,), jnp.int32),
            pltpu.VMEM((CHUNK, 2 * EMB_DIM), jnp.float32),
            pltpu.VMEM((CHUNK, 2 * EMB_DIM), jnp.float32),
            pltpu.VMEM((SEQ_LEN, EMB_DIM), jnp.float32),
            pltpu.VMEM((SEQ_LEN, EMB_DIM), jnp.float32),
            pltpu.VMEM((MAX_LEN * EMB_DIM,), jnp.float32),
            pltpu.SemaphoreType.DMA,
            pltpu.SemaphoreType.DMA,
            pltpu.SemaphoreType.DMA,
            pltpu.SemaphoreType.DMA,
            pltpu.SemaphoreType.DMA,
            pltpu.SemaphoreType.DMA,
        ],
    )(x_flat, wt_wide, pos_flat)
    return out
